# baseline (speedup 1.0000x reference)
import functools

import jax
import jax.numpy as jnp
from jax import lax
from jax.experimental import pallas as pl
from jax.experimental.pallas import tpu as pltpu
from jax.experimental.pallas import tpu_sc as plsc

NC = 2
NT = 16
L = 16
CHUNK = 128
CW = 16
TRASH = 256
BLK = 1000
G = 256


def _embed_body(xu, xi, wu, bu, wi, bi, hu, hi):
    hu[...] = jnp.dot(xu[...], wu[...], preferred_element_type=jnp.float32) + bu[...]
    hi[...] = jnp.dot(xi[...], wi[...], preferred_element_type=jnp.float32) + bi[...]


def _embed(x_user, x_item, W_emb_u, b_emb_u, W_emb_i, b_emb_i):
    n, d = x_user.shape
    h = W_emb_u.shape[1]
    nblk = n // BLK
    row = lambda j: (j, 0)
    full = lambda j: (0, 0)
    return pl.pallas_call(
        _embed_body,
        grid=(nblk,),
        in_specs=[
            pl.BlockSpec((BLK, d), row),
            pl.BlockSpec((BLK, d), row),
            pl.BlockSpec((d, h), full),
            pl.BlockSpec((1, h), full),
            pl.BlockSpec((d, h), full),
            pl.BlockSpec((1, h), full),
        ],
        out_specs=[pl.BlockSpec((BLK, h), row)] * 2,
        out_shape=[jax.ShapeDtypeStruct((n, h), jnp.float32)] * 2,
        compiler_params=pltpu.CompilerParams(
            dimension_semantics=("arbitrary",)),
    )(x_user, x_item, W_emb_u, b_emb_u.reshape(1, h), W_emb_i,
      b_emb_i.reshape(1, h))


def _remap_body(half, dui, diu, out):
    res = []
    for d in (dui[...], diu[...]):
        for c in range(NC):
            lo = c * half
            inb = (d >= lo) & (d < lo + half)
            res.append(jnp.where(inb, d - lo, half + (d & (TRASH - 1))))
    out[...] = jnp.stack(res, axis=0)


def _remap_dst(dst_ui_flat, dst_iu_flat, half):
    epad = dst_ui_flat.shape[0]
    w = 512
    r = epad // w
    body = functools.partial(_remap_body, half)
    out = pl.pallas_call(
        body,
        in_specs=[pl.BlockSpec((r, w), lambda: (0, 0))] * 2,
        out_specs=pl.BlockSpec((2 * NC, r, w), lambda: (0, 0, 0)),
        out_shape=jax.ShapeDtypeStruct((2 * NC, r, w), jnp.int32),
    )(dst_ui_flat.reshape(r, w), dst_iu_flat.reshape(r, w))
    return out.reshape(2 * NC, epad)


def _sc_segment_sums(h_u, h_i, src_ui, dst_ui, src_iu, dst_iu, rows_pad):
    n, h = h_u.shape
    nchunk = src_ui.shape[0] // (NT * CHUNK)
    half = rows_pad // NC
    arows = half + TRASH
    zpt = arows // NT
    fpt = half // NT
    zacc = jnp.zeros((arows, h), jnp.float32)
    ones_h = jnp.ones((CHUNK, h), jnp.float32)

    mesh = plsc.VectorSubcoreMesh(core_axis_name="c", subcore_axis_name="s")

    @functools.partial(
        pl.kernel,
        mesh=mesh,
        out_type=[
            jax.ShapeDtypeStruct((rows_pad, h), jnp.float32),
            jax.ShapeDtypeStruct((rows_pad, h), jnp.float32),
            jax.ShapeDtypeStruct((rows_pad, h), jnp.float32),
            jax.ShapeDtypeStruct((rows_pad, h), jnp.float32),
        ],
        scratch_types=[
            pltpu.VMEM((2 * CHUNK,), jnp.int32),
            pltpu.VMEM((2 * CHUNK,), jnp.int32),
            pltpu.VMEM((2 * CHUNK,), jnp.int32),
            pltpu.VMEM((2 * CHUNK,), jnp.int32),
            pltpu.VMEM((2 * CHUNK, h), jnp.float32),
            pltpu.VMEM((2 * CHUNK, h), jnp.float32),
            pltpu.VMEM_SHARED((arows, h), jnp.float32),
            pltpu.SemaphoreType.DMA,
            pltpu.SemaphoreType.DMA,
            pltpu.SemaphoreType.DMA,
            pltpu.SemaphoreType.DMA,
        ],
    )
    def k(hu, hi, sui, dui, siu, diu, zacc_h, ones_hbm,
          sum_i, sum_u, cnt_i, cnt_u,
          sidx0, sidx1, didx0, didx1, rows0, rows1, acc,
          sem0, sem1, semd0, semd1):
        cid = lax.axis_index("c")
        sid = lax.axis_index("s")
        lo = cid * half
        zsl = pl.ds(sid * zpt, zpt)
        nmac = nchunk // 2
        n2 = nmac // 2

        def zero():
            pltpu.sync_copy(zacc_h.at[zsl], acc.at[zsl])

        def epilogue(out):
            plsc.subcore_barrier()
            fsl = pl.ds(sid * fpt, fpt)
            osl = pl.ds(lo + sid * fpt, fpt)
            pltpu.sync_copy(acc.at[fsl], out.at[osl])
            plsc.subcore_barrier()
            zero()
            plsc.subcore_barrier()

        def sum_phase(h_ref, src_ref, dst_ref, out):
            sbase = sid * (nchunk * CHUNK)
            dbase = (cid * NT + sid) * (nchunk * CHUNK)

            def load_idx(ref, base_el, j, buf):
                for q in range(2):
                    pltpu.sync_copy(
                        ref.at[pl.ds(base_el + (2 * j + q) * CHUNK, CHUNK)],
                        buf.at[pl.ds(q * CHUNK, CHUNK)])

            load_idx(src_ref, sbase, 0, sidx0)
            pltpu.async_copy(h_ref.at[sidx0], rows0, sem0)

            def step(j2, carry):
                base = 2 * j2
                load_idx(src_ref, sbase, base + 1, sidx1)
                pltpu.async_copy(h_ref.at[sidx1], rows1, sem1)
                load_idx(dst_ref, dbase, base, didx0)
                pltpu.make_async_copy(h_ref.at[sidx0], rows0, sem0).wait()
                pltpu.sync_copy(rows0, acc.at[didx0], add=True)
                nxt = jnp.where(base + 2 >= nmac, 0, base + 2)
                load_idx(src_ref, sbase, nxt, sidx0)
                pltpu.async_copy(h_ref.at[sidx0], rows0, sem0)
                load_idx(dst_ref, dbase, base + 1, didx1)
                pltpu.make_async_copy(h_ref.at[sidx1], rows1, sem1).wait()
                pltpu.sync_copy(rows1, acc.at[didx1], add=True)
                return carry

            lax.fori_loop(0, n2, step, 0)
            pltpu.make_async_copy(h_ref.at[sidx0], rows0, sem0).wait()
            epilogue(out)

        def cnt_phase(dst_ref, out):
            dbase = (cid * NT + sid) * (nchunk * CHUNK)
            pltpu.sync_copy(ones_hbm, rows0.at[pl.ds(0, CHUNK)])
            pltpu.sync_copy(ones_hbm, rows0.at[pl.ds(CHUNK, CHUNK)])

            def load_idx(j, buf):
                for q in range(2):
                    pltpu.sync_copy(
                        dst_ref.at[pl.ds(dbase + (2 * j + q) * CHUNK, CHUNK)],
                        buf.at[pl.ds(q * CHUNK, CHUNK)])

            load_idx(0, didx0)

            def step(j2, carry):
                base = 2 * j2
                load_idx(base + 1, didx1)
                pltpu.sync_copy(rows0, acc.at[didx0], add=True)
                nxt = jnp.where(base + 2 >= nmac, 0, base + 2)
                load_idx(nxt, didx0)
                pltpu.sync_copy(rows0, acc.at[didx1], add=True)
                return carry

            lax.fori_loop(0, n2, step, 0)
            epilogue(out)

        zero()
        plsc.subcore_barrier()
        sum_phase(hu, sui, dui, sum_i)
        cnt_phase(dui, cnt_i)
        sum_phase(hi, siu, diu, sum_u)
        cnt_phase(diu, cnt_u)

    return k(h_u, h_i, src_ui, dst_ui, src_iu, dst_iu, zacc, ones_h)


def _combine_body(si, ci, hi, bti, su, cu, hu, btu,
                  wl_ui, bl_ui, wr_ui, wl_iu, bl_iu, wr_iu, wo, bo,
                  out, pu_acc, pi_acc, ncu_acc, nci_acc):
    j = pl.program_id(0)
    nblk = pl.num_programs(0)

    @pl.when(j == 0)
    def _():
        pu_acc[...] = jnp.zeros_like(pu_acc)
        pi_acc[...] = jnp.zeros_like(pi_acc)
        ncu_acc[...] = jnp.zeros_like(ncu_acc)
        nci_acc[...] = jnp.zeros_like(nci_acc)

    f32 = jnp.float32
    dot = functools.partial(jnp.dot, preferred_element_type=f32)
    agg_i = si[...] / jnp.maximum(ci[...][:, 0:1], 1.0)
    out_i = dot(agg_i, wl_ui[...]) + bl_ui[...] + dot(hi[...], wr_ui[...])
    h_i2 = jnp.maximum(out_i, 0.0)
    agg_u = su[...] / jnp.maximum(cu[...][:, 0:1], 1.0)
    out_u = dot(agg_u, wl_iu[...]) + bl_iu[...] + dot(hu[...], wr_iu[...])
    h_u2 = jnp.maximum(out_u, 0.0)

    nrows = h_u2.shape[0]
    gids = lax.broadcasted_iota(jnp.int32, (nrows, G), 1)
    ct = (((0,), (0,)), ((), ()))
    oh_u = (btu[...] == gids).astype(f32)
    oh_i = (bti[...] == gids).astype(f32)
    ones_blk = jnp.ones((nrows, pu_acc.shape[1]), f32)
    pu_acc[...] += lax.dot_general(oh_u, h_u2, ct, preferred_element_type=f32)
    pi_acc[...] += lax.dot_general(oh_i, h_i2, ct, preferred_element_type=f32)
    ncu_acc[...] += lax.dot_general(oh_u, ones_blk, ct, preferred_element_type=f32)
    nci_acc[...] += lax.dot_general(oh_i, ones_blk, ct, preferred_element_type=f32)

    @pl.when(j == nblk - 1)
    def _():
        p_u = pu_acc[...] / jnp.maximum(ncu_acc[...], 1.0)
        p_i = pi_acc[...] / jnp.maximum(nci_acc[...], 1.0)
        x = 0.5 * (p_u + p_i)
        out[...] = dot(x, wo[...]) + bo[...]


def _combine(sum_i, cnt_i, h_i, batch_i, sum_u, cnt_u, h_u, batch_u,
             Wl_ui, bl_ui, Wr_ui, Wl_iu, bl_iu, Wr_iu, W_out, b_out):
    n, h = h_u.shape
    o = W_out.shape[1]
    nblk = n // BLK
    row = lambda j: (j, 0)
    full = lambda j: (0, 0)
    return pl.pallas_call(
        _combine_body,
        grid=(nblk,),
        in_specs=[
            pl.BlockSpec((BLK, h), row),
            pl.BlockSpec((BLK, h), row),
            pl.BlockSpec((BLK, h), row),
            pl.BlockSpec((BLK, 1), row),
            pl.BlockSpec((BLK, h), row),
            pl.BlockSpec((BLK, h), row),
            pl.BlockSpec((BLK, h), row),
            pl.BlockSpec((BLK, 1), row),
            pl.BlockSpec((h, h), full),
            pl.BlockSpec((1, h), full),
            pl.BlockSpec((h, h), full),
            pl.BlockSpec((h, h), full),
            pl.BlockSpec((1, h), full),
            pl.BlockSpec((h, h), full),
            pl.BlockSpec((h, o), full),
            pl.BlockSpec((1, o), full),
        ],
        out_specs=pl.BlockSpec((G, o), full),
        out_shape=jax.ShapeDtypeStruct((G, o), jnp.float32),
        scratch_shapes=[
            pltpu.VMEM((G, h), jnp.float32),
            pltpu.VMEM((G, h), jnp.float32),
            pltpu.VMEM((G, h), jnp.float32),
            pltpu.VMEM((G, h), jnp.float32),
        ],
        compiler_params=pltpu.CompilerParams(
            dimension_semantics=("arbitrary",)),
    )(sum_i, cnt_i, h_i, batch_i, sum_u, cnt_u, h_u, batch_u,
      Wl_ui, bl_ui.reshape(1, h), Wr_ui, Wl_iu, bl_iu.reshape(1, h), Wr_iu,
      W_out, b_out.reshape(1, o))


def kernel(x_user, x_item, edge_index_ui, edge_index_iu, edge_attr_ui,
           edge_attr_iu, batch_user, batch_item, W_emb_u, b_emb_u, W_emb_i,
           b_emb_i, Wl_ui, bl_ui, Wr_ui, Wl_iu, bl_iu, Wr_iu, W_out, b_out):
    n = x_user.shape[0]
    e = edge_index_ui.shape[1]
    rows_pad = -(-(n + 1) // (NC * NT * 16)) * (NC * NT * 16)
    nchunk = -(-e // (NT * CHUNK))
    nchunk = -(-nchunk // 8) * 8
    ept = nchunk * CHUNK
    epad = ept * NT

    src_ui = jnp.pad(edge_index_ui[0], (0, epad - e))
    src_iu = jnp.pad(edge_index_iu[0], (0, epad - e))
    dui_flat = jnp.pad(edge_index_ui[1], (0, epad - e), constant_values=n)
    diu_flat = jnp.pad(edge_index_iu[1], (0, epad - e), constant_values=n)
    dmap = _remap_dst(dui_flat, diu_flat, rows_pad // NC)
    dst_ui = dmap[:NC].reshape(NC * epad)
    dst_iu = dmap[NC:].reshape(NC * epad)

    h_u, h_i = _embed(x_user, x_item, W_emb_u, b_emb_u, W_emb_i, b_emb_i)
    sum_i, sum_u, cnt_i, cnt_u = _sc_segment_sums(
        h_u, h_i, src_ui, dst_ui, src_iu, dst_iu, rows_pad)
    return _combine(sum_i, cnt_i, h_i, batch_item.reshape(n, 1),
                    sum_u, cnt_u, h_u, batch_user.reshape(n, 1),
                    Wl_ui, bl_ui, Wr_ui, Wl_iu, bl_iu, Wr_iu, W_out, b_out)


# speedup vs baseline: 1.0691x; 1.0691x over previous
import functools

import jax
import jax.numpy as jnp
from jax import lax
from jax.experimental import pallas as pl
from jax.experimental.pallas import tpu as pltpu
from jax.experimental.pallas import tpu_sc as plsc

NC = 2
NT = 16
L = 16
CHUNK = 128
CW = 16
TRASH = 256
BLK = 1000
G = 256


def _embed_body(xu, xi, wu, bu, wi, bi, hu, hi):
    hu[...] = jnp.dot(xu[...], wu[...], preferred_element_type=jnp.float32) + bu[...]
    hi[...] = jnp.dot(xi[...], wi[...], preferred_element_type=jnp.float32) + bi[...]


def _embed(x_user, x_item, W_emb_u, b_emb_u, W_emb_i, b_emb_i):
    n, d = x_user.shape
    h = W_emb_u.shape[1]
    nblk = n // BLK
    row = lambda j: (j, 0)
    full = lambda j: (0, 0)
    return pl.pallas_call(
        _embed_body,
        grid=(nblk,),
        in_specs=[
            pl.BlockSpec((BLK, d), row),
            pl.BlockSpec((BLK, d), row),
            pl.BlockSpec((d, h), full),
            pl.BlockSpec((1, h), full),
            pl.BlockSpec((d, h), full),
            pl.BlockSpec((1, h), full),
        ],
        out_specs=[pl.BlockSpec((BLK, h), row)] * 2,
        out_shape=[jax.ShapeDtypeStruct((n, h), jnp.float32)] * 2,
        compiler_params=pltpu.CompilerParams(
            dimension_semantics=("arbitrary",)),
    )(x_user, x_item, W_emb_u, b_emb_u.reshape(1, h), W_emb_i,
      b_emb_i.reshape(1, h))


def _remap_body(half, dui, diu, out):
    res = []
    for d in (dui[...], diu[...]):
        for c in range(NC):
            lo = c * half
            inb = (d >= lo) & (d < lo + half)
            res.append(jnp.where(inb, d - lo, half + (d & (TRASH - 1))))
    out[...] = jnp.stack(res, axis=0)


def _remap_dst(dst_ui_flat, dst_iu_flat, half):
    epad = dst_ui_flat.shape[0]
    w = 512
    r = epad // w
    body = functools.partial(_remap_body, half)
    out = pl.pallas_call(
        body,
        in_specs=[pl.BlockSpec((r, w), lambda: (0, 0))] * 2,
        out_specs=pl.BlockSpec((2 * NC, r, w), lambda: (0, 0, 0)),
        out_shape=jax.ShapeDtypeStruct((2 * NC, r, w), jnp.int32),
    )(dst_ui_flat.reshape(r, w), dst_iu_flat.reshape(r, w))
    return out.reshape(2 * NC, epad)


def _sc_segment_sums(h_u, h_i, src_ui, dst_ui, src_iu, dst_iu, rows_pad):
    n, h = h_u.shape
    nchunk = src_ui.shape[0] // (NT * CHUNK)
    half = rows_pad // NC
    arows = half + TRASH
    zpt = arows // NT
    fpt = half // NT
    zacc = jnp.zeros((arows, h), jnp.float32)
    ones_h = jnp.ones((CHUNK, h), jnp.float32)

    mesh = plsc.VectorSubcoreMesh(core_axis_name="c", subcore_axis_name="s")

    @functools.partial(
        pl.kernel,
        mesh=mesh,
        out_type=[
            jax.ShapeDtypeStruct((rows_pad, h), jnp.float32),
            jax.ShapeDtypeStruct((rows_pad, h), jnp.float32),
            jax.ShapeDtypeStruct((rows_pad, h), jnp.float32),
            jax.ShapeDtypeStruct((rows_pad, h), jnp.float32),
        ],
        scratch_types=[
            pltpu.VMEM((CHUNK,), jnp.int32),
            pltpu.VMEM((CHUNK,), jnp.int32),
            pltpu.VMEM((CHUNK,), jnp.int32),
            pltpu.VMEM((CHUNK,), jnp.int32),
            pltpu.VMEM((CHUNK, h), jnp.float32),
            pltpu.VMEM((CHUNK, h), jnp.float32),
            pltpu.VMEM_SHARED((arows, h), jnp.float32),
            pltpu.SemaphoreType.DMA,
            pltpu.SemaphoreType.DMA,
            pltpu.SemaphoreType.DMA,
            pltpu.SemaphoreType.DMA,
        ],
    )
    def k(hu, hi, sui, dui, siu, diu, zacc_h, ones_hbm,
          sum_i, sum_u, cnt_i, cnt_u,
          sidx0, sidx1, didx0, didx1, rows0, rows1, acc,
          sem0, sem1, semd0, semd1):
        cid = lax.axis_index("c")
        sid = lax.axis_index("s")
        lo = cid * half
        zsl = pl.ds(sid * zpt, zpt)
        n2 = nchunk // 2

        def zero():
            pltpu.sync_copy(zacc_h.at[zsl], acc.at[zsl])

        def epilogue(out):
            plsc.subcore_barrier()
            fsl = pl.ds(sid * fpt, fpt)
            osl = pl.ds(lo + sid * fpt, fpt)
            pltpu.sync_copy(acc.at[fsl], out.at[osl])
            plsc.subcore_barrier()
            zero()
            plsc.subcore_barrier()

        def sum_phase(h_ref, src_ref, dst_ref, out):
            sbase = sid * (nchunk * CHUNK)
            dbase = (cid * NT + sid) * (nchunk * CHUNK)

            def sslice(j):
                return src_ref.at[pl.ds(sbase + j * CHUNK, CHUNK)]

            def dslice(j):
                return dst_ref.at[pl.ds(dbase + j * CHUNK, CHUNK)]

            pltpu.sync_copy(sslice(0), sidx0)
            pltpu.async_copy(h_ref.at[sidx0], rows0, sem0)

            def step(j2, carry):
                base = 2 * j2
                pltpu.sync_copy(sslice(base + 1), sidx1)
                pltpu.async_copy(h_ref.at[sidx1], rows1, sem1)
                pltpu.sync_copy(dslice(base), didx0)
                pltpu.make_async_copy(h_ref.at[sidx0], rows0, sem0).wait()
                pltpu.sync_copy(rows0, acc.at[didx0], add=True)
                nxt = jnp.where(base + 2 >= nchunk, 0, base + 2)
                pltpu.sync_copy(
                    src_ref.at[pl.ds(sbase + nxt * CHUNK, CHUNK)], sidx0)
                pltpu.async_copy(h_ref.at[sidx0], rows0, sem0)
                pltpu.sync_copy(dslice(base + 1), didx1)
                pltpu.make_async_copy(h_ref.at[sidx1], rows1, sem1).wait()
                pltpu.sync_copy(rows1, acc.at[didx1], add=True)
                return carry

            lax.fori_loop(0, n2, step, 0)
            pltpu.make_async_copy(h_ref.at[sidx0], rows0, sem0).wait()
            epilogue(out)

        def cnt_phase(dst_ref, out):
            dbase = (cid * NT + sid) * (nchunk * CHUNK)
            pltpu.sync_copy(ones_hbm, rows0)

            def dslice(j):
                return dst_ref.at[pl.ds(dbase + j * CHUNK, CHUNK)]

            pltpu.async_copy(dslice(0), didx0, semd0)

            def step(j2, carry):
                base = 2 * j2
                pltpu.async_copy(dslice(base + 1), didx1, semd1)
                pltpu.make_async_copy(dslice(base), didx0, semd0).wait()
                pltpu.sync_copy(rows0, acc.at[didx0], add=True)
                nxt = jnp.where(base + 2 >= nchunk, 0, base + 2)
                pltpu.async_copy(dslice(nxt), didx0, semd0)
                pltpu.make_async_copy(dslice(base + 1), didx1, semd1).wait()
                pltpu.sync_copy(rows0, acc.at[didx1], add=True)
                return carry

            lax.fori_loop(0, n2, step, 0)
            pltpu.make_async_copy(dslice(0), didx0, semd0).wait()
            epilogue(out)

        zero()
        plsc.subcore_barrier()
        sum_phase(hu, sui, dui, sum_i)
        cnt_phase(dui, cnt_i)
        sum_phase(hi, siu, diu, sum_u)
        cnt_phase(diu, cnt_u)

    return k(h_u, h_i, src_ui, dst_ui, src_iu, dst_iu, zacc, ones_h)


def _combine_body(si, ci, hi, bti, su, cu, hu, btu,
                  wl_ui, bl_ui, wr_ui, wl_iu, bl_iu, wr_iu, wo, bo,
                  out, pu_acc, pi_acc, ncu_acc, nci_acc):
    j = pl.program_id(0)
    nblk = pl.num_programs(0)

    @pl.when(j == 0)
    def _():
        pu_acc[...] = jnp.zeros_like(pu_acc)
        pi_acc[...] = jnp.zeros_like(pi_acc)
        ncu_acc[...] = jnp.zeros_like(ncu_acc)
        nci_acc[...] = jnp.zeros_like(nci_acc)

    f32 = jnp.float32
    dot = functools.partial(jnp.dot, preferred_element_type=f32)
    agg_i = si[...] / jnp.maximum(ci[...][:, 0:1], 1.0)
    out_i = dot(agg_i, wl_ui[...]) + bl_ui[...] + dot(hi[...], wr_ui[...])
    h_i2 = jnp.maximum(out_i, 0.0)
    agg_u = su[...] / jnp.maximum(cu[...][:, 0:1], 1.0)
    out_u = dot(agg_u, wl_iu[...]) + bl_iu[...] + dot(hu[...], wr_iu[...])
    h_u2 = jnp.maximum(out_u, 0.0)

    nrows = h_u2.shape[0]
    gids = lax.broadcasted_iota(jnp.int32, (nrows, G), 1)
    ct = (((0,), (0,)), ((), ()))
    oh_u = (btu[...] == gids).astype(f32)
    oh_i = (bti[...] == gids).astype(f32)
    ones_blk = jnp.ones((nrows, pu_acc.shape[1]), f32)
    pu_acc[...] += lax.dot_general(oh_u, h_u2, ct, preferred_element_type=f32)
    pi_acc[...] += lax.dot_general(oh_i, h_i2, ct, preferred_element_type=f32)
    ncu_acc[...] += lax.dot_general(oh_u, ones_blk, ct, preferred_element_type=f32)
    nci_acc[...] += lax.dot_general(oh_i, ones_blk, ct, preferred_element_type=f32)

    @pl.when(j == nblk - 1)
    def _():
        p_u = pu_acc[...] / jnp.maximum(ncu_acc[...], 1.0)
        p_i = pi_acc[...] / jnp.maximum(nci_acc[...], 1.0)
        x = 0.5 * (p_u + p_i)
        out[...] = dot(x, wo[...]) + bo[...]


def _combine(sum_i, cnt_i, h_i, batch_i, sum_u, cnt_u, h_u, batch_u,
             Wl_ui, bl_ui, Wr_ui, Wl_iu, bl_iu, Wr_iu, W_out, b_out):
    n, h = h_u.shape
    o = W_out.shape[1]
    nblk = n // BLK
    row = lambda j: (j, 0)
    full = lambda j: (0, 0)
    return pl.pallas_call(
        _combine_body,
        grid=(nblk,),
        in_specs=[
            pl.BlockSpec((BLK, h), row),
            pl.BlockSpec((BLK, h), row),
            pl.BlockSpec((BLK, h), row),
            pl.BlockSpec((BLK, 1), row),
            pl.BlockSpec((BLK, h), row),
            pl.BlockSpec((BLK, h), row),
            pl.BlockSpec((BLK, h), row),
            pl.BlockSpec((BLK, 1), row),
            pl.BlockSpec((h, h), full),
            pl.BlockSpec((1, h), full),
            pl.BlockSpec((h, h), full),
            pl.BlockSpec((h, h), full),
            pl.BlockSpec((1, h), full),
            pl.BlockSpec((h, h), full),
            pl.BlockSpec((h, o), full),
            pl.BlockSpec((1, o), full),
        ],
        out_specs=pl.BlockSpec((G, o), full),
        out_shape=jax.ShapeDtypeStruct((G, o), jnp.float32),
        scratch_shapes=[
            pltpu.VMEM((G, h), jnp.float32),
            pltpu.VMEM((G, h), jnp.float32),
            pltpu.VMEM((G, h), jnp.float32),
            pltpu.VMEM((G, h), jnp.float32),
        ],
        compiler_params=pltpu.CompilerParams(
            dimension_semantics=("arbitrary",)),
    )(sum_i, cnt_i, h_i, batch_i, sum_u, cnt_u, h_u, batch_u,
      Wl_ui, bl_ui.reshape(1, h), Wr_ui, Wl_iu, bl_iu.reshape(1, h), Wr_iu,
      W_out, b_out.reshape(1, o))


def kernel(x_user, x_item, edge_index_ui, edge_index_iu, edge_attr_ui,
           edge_attr_iu, batch_user, batch_item, W_emb_u, b_emb_u, W_emb_i,
           b_emb_i, Wl_ui, bl_ui, Wr_ui, Wl_iu, bl_iu, Wr_iu, W_out, b_out):
    n = x_user.shape[0]
    e = edge_index_ui.shape[1]
    rows_pad = -(-(n + 1) // (NC * NT * 16)) * (NC * NT * 16)
    nchunk = -(-e // (NT * CHUNK))
    nchunk = -(-nchunk // 8) * 8
    ept = nchunk * CHUNK
    epad = ept * NT

    src_ui = jnp.pad(edge_index_ui[0], (0, epad - e))
    src_iu = jnp.pad(edge_index_iu[0], (0, epad - e))
    dui_flat = jnp.pad(edge_index_ui[1], (0, epad - e), constant_values=n)
    diu_flat = jnp.pad(edge_index_iu[1], (0, epad - e), constant_values=n)
    dmap = _remap_dst(dui_flat, diu_flat, rows_pad // NC)
    dst_ui = dmap[:NC].reshape(NC * epad)
    dst_iu = dmap[NC:].reshape(NC * epad)

    h_u, h_i = _embed(x_user, x_item, W_emb_u, b_emb_u, W_emb_i, b_emb_i)
    sum_i, sum_u, cnt_i, cnt_u = _sc_segment_sums(
        h_u, h_i, src_ui, dst_ui, src_iu, dst_iu, rows_pad)
    return _combine(sum_i, cnt_i, h_i, batch_item.reshape(n, 1),
                    sum_u, cnt_u, h_u, batch_user.reshape(n, 1),
                    Wl_ui, bl_ui, Wr_ui, Wl_iu, bl_iu, Wr_iu, W_out, b_out)
